# Initial kernel scaffold; baseline (speedup 1.0000x reference)
import jax
import jax.numpy as jnp
from jax.experimental import pallas as pl


def kernel(x, row_table, col_table, io_table, pair_table):
    raise NotImplementedError("write your pallas kernel here")


# speedup vs baseline: 2.3593x; 2.3593x over previous
import jax
import jax.numpy as jnp
from jax.experimental import pallas as pl


def _body(row_ref, col_ref, io_ref, pair_ref, out_ref):
    g = pl.program_id(0)
    hb = out_ref.shape[1]
    w = col_ref.shape[0]
    d4 = row_ref.shape[1]
    row = row_ref[...]
    col = col_ref[...]
    io_row = io_ref[pl.ds(g % 2, 1), :]
    pair_row = pair_ref[pl.ds(g // 2, 1), :]
    tile = jnp.concatenate(
        [
            jnp.broadcast_to(row[:, None, :], (hb, w, d4)),
            jnp.broadcast_to(col[None, :, :], (hb, w, d4)),
            jnp.broadcast_to(io_row[None, :, :], (hb, w, d4)),
            jnp.broadcast_to(pair_row[None, :, :], (hb, w, d4)),
        ],
        axis=-1,
    )
    out_ref[...] = tile[None]


def kernel(x, row_table, col_table, io_table, pair_table):
    _, num_grids, height, width, d_model = x.shape
    d4 = d_model // 4
    hb = 8
    grid = (num_grids, height // hb)
    out = pl.pallas_call(
        _body,
        grid=grid,
        in_specs=[
            pl.BlockSpec((hb, d4), lambda g, h: (h, 0)),
            pl.BlockSpec((width, d4), lambda g, h: (0, 0)),
            pl.BlockSpec(io_table.shape, lambda g, h: (0, 0)),
            pl.BlockSpec(pair_table.shape, lambda g, h: (0, 0)),
        ],
        out_specs=pl.BlockSpec((1, hb, width, d_model), lambda g, h: (g, h, 0, 0)),
        out_shape=jax.ShapeDtypeStruct((num_grids, height, width, d_model), jnp.float32),
    )(row_table, col_table, io_table, pair_table)
    return out
